# baseline (speedup 1.0000x reference)
import functools

import jax
import jax.numpy as jnp
from jax import lax
from jax.experimental import pallas as pl
from jax.experimental.pallas import tpu as pltpu
from jax.experimental.pallas import tpu_sc as plsc

B = 4096
L = 200
DIM = 32
VOCAB = 1000000

NUM_CORES = 2
NUM_SUBCORES = 16
NW = NUM_CORES * NUM_SUBCORES

BCHUNK = 512
NCHUNK = B // BCHUNK
NBLK_TOTAL = L * NCHUNK
BLK_PER_W = NBLK_TOTAL // NW
NPAIR = BLK_PER_W // 2
NTR = DIM // 8
NTC = BCHUNK // 128
PITCH = DIM + 1

NTILE = 7816
VPAD = NTILE * 128
TBLK = 8
NTBLK = NTILE // TBLK
TP_PER_W = 31
OPITCH = DIM + 2


def _sc_table_transpose(in6):
    mesh = plsc.VectorSubcoreMesh(core_axis_name="c", subcore_axis_name="s")

    @functools.partial(
        pl.kernel,
        mesh=mesh,
        out_type=jax.ShapeDtypeStruct((VPAD, DIM), jnp.float32),
        compiler_params=pltpu.CompilerParams(
            use_tc_tiling_on_sc=False, needs_layout_passes=False),
        scratch_types=[
            pltpu.VMEM((2, NTR, TBLK, 8, 128), jnp.float32),
            pltpu.VMEM((TBLK * 128, OPITCH), jnp.float32),
            pltpu.SemaphoreType.DMA,
            pltpu.SemaphoreType.DMA,
        ],
    )
    def body(in_hbm, out_hbm, sbuf_v, obuf_v, sem0, sem1):
        wid = lax.axis_index("s") * NUM_CORES + lax.axis_index("c")
        sems = (sem0, sem1)

        def issue_in(bid, buf):
            @pl.when(bid < NTBLK)
            def _():
                for tr in range(NTR):
                    pltpu.async_copy(in_hbm.at[tr, pl.ds(bid * TBLK, TBLK)],
                                     sbuf_v.at[buf].at[tr], sems[buf])

        def wait_in(bid, buf):
            for tr in range(NTR):
                pltpu.make_async_copy(
                    in_hbm.at[tr, pl.ds(bid * TBLK, TBLK)],
                    sbuf_v.at[buf].at[tr], sems[buf]).wait()

        def do_block(bid, buf):
            wait_in(bid, buf)

            @plsc.parallel_loop(0, NTR * TBLK * 8, unroll=4)
            def _(it):
                tr = it // (TBLK * 8)
                tcr = (it // 8) % TBLK
                r = it % 8
                d = tr * 8 + r
                dsplat = jnp.zeros((16,), jnp.int32) + d
                for c8 in range(8):
                    c0 = c8 * 16
                    v = sbuf_v[buf, tr, tcr, r, pl.ds(c0, 16)]
                    tvec = tcr * 128 + c0 + lax.iota(jnp.int32, 16)
                    plsc.store_scatter(obuf_v, [tvec, dsplat], v)

            pltpu.sync_copy(
                obuf_v.at[pl.ds(0, TBLK * 128), pl.ds(0, DIM)],
                out_hbm.at[pl.ds(bid * TBLK * 128, TBLK * 128)])

        issue_in(wid, 0)

        for k in range(TP_PER_W):
            bid = wid + NW * k
            if k + 1 < TP_PER_W:
                issue_in(bid + NW, (k + 1) % 2)

            @pl.when(bid < NTBLK)
            def _():
                do_block(bid, k % 2)

    return body(in6)


def _sc_embed(xt, table, pos_table):
    mesh = plsc.VectorSubcoreMesh(core_axis_name="c", subcore_axis_name="s")

    @functools.partial(
        pl.kernel,
        mesh=mesh,
        out_type=jax.ShapeDtypeStruct((L, NTR, B // 128, 8, 128), jnp.float32),
        compiler_params=pltpu.CompilerParams(
            use_tc_tiling_on_sc=False, needs_layout_passes=False),
        scratch_types=[
            pltpu.VMEM((2, BCHUNK), jnp.int32),
            pltpu.VMEM((2, BCHUNK), jnp.float32),
            pltpu.VMEM((2, BCHUNK, DIM), jnp.float32),
            pltpu.VMEM((BCHUNK, PITCH), jnp.float32),
            pltpu.VMEM((2, NTR, NTC, 8, 128), jnp.float32),
            pltpu.VMEM((L, DIM), jnp.float32),
            pltpu.VMEM((DIM, 16), jnp.float32),
            pltpu.SemaphoreType.DMA,
            pltpu.SemaphoreType.DMA,
            pltpu.SemaphoreType.DMA,
            pltpu.SemaphoreType.DMA,
        ],
    )
    def body(x_hbm, emb_hbm, pos_hbm, out_hbm, tok_v, fm_v, rows_v,
             st_v, tbuf_v, pos_v, psplat_v, gsem0, gsem1, osem0, osem1):
        wid = lax.axis_index("s") * NUM_CORES + lax.axis_index("c")
        bid0 = wid * BLK_PER_W
        pltpu.sync_copy(pos_hbm.at[pl.ds(0, L)], pos_v)
        gsems = (gsem0, gsem1)
        osems = (osem0, osem1)

        def fetch(k, buf):
            bid = bid0 + k
            l = bid // NCHUNK
            b0 = (bid % NCHUNK) * BCHUNK
            pltpu.sync_copy(x_hbm.at[l, pl.ds(b0, BCHUNK)], tok_v.at[buf])

            @plsc.parallel_loop(0, BCHUNK, step=16, unroll=4)
            def _(i):
                iv = tok_v[buf, pl.ds(i, 16)]
                fm_v[buf, pl.ds(i, 16)] = jnp.where(
                    iv == jnp.int32(0), jnp.float32(0.0), jnp.float32(1.0))

            return pltpu.async_copy(
                emb_hbm.at[tok_v.at[buf]], rows_v.at[buf], gsems[buf])

        def process(k, buf):
            bid = bid0 + k
            l = bid // NCHUNK
            chunk = bid % NCHUNK

            lsplat = jnp.zeros((16,), jnp.int32) + l
            for d in range(DIM):
                psplat_v[d] = plsc.load_gather(
                    pos_v, [lsplat, jnp.full((16,), d, jnp.int32)])

            @plsc.parallel_loop(0, BCHUNK, step=4, unroll=4)
            def _(r):
                for u in range(4):
                    st_v[r + u, pl.ds(0, 16)] = rows_v[buf, r + u, pl.ds(0, 16)]
                    st_v[r + u, pl.ds(16, 16)] = (
                        rows_v[buf, r + u, pl.ds(16, 16)])

            @plsc.parallel_loop(0, BCHUNK // 16, unroll=4)
            def _(i16):
                i = i16 * 16
                rvec = i + lax.iota(jnp.int32, 16)
                fm = fm_v[buf, pl.ds(i, 16)]
                tc = i16 // 8
                c0 = (i16 % 8) * 16
                for d in range(DIM):
                    val = plsc.load_gather(
                        st_v, [rvec, jnp.full((16,), d, jnp.int32)])
                    tbuf_v[buf, d // 8, tc, d % 8, pl.ds(c0, 16)] = (
                        val + psplat_v[d] * fm)

            return [
                pltpu.async_copy(
                    tbuf_v.at[buf].at[tr],
                    out_hbm.at[l, tr, pl.ds(chunk * NTC, NTC)],
                    osems[buf])
                for tr in range(NTR)
            ]

        cp0 = fetch(0, 0)

        @pl.loop(0, NPAIR)
        def _(j):
            k0 = j * 2
            cp1 = fetch(k0 + 1, 1)
            cp0 = pltpu.make_async_copy(
                emb_hbm.at[tok_v.at[0]], rows_v.at[0], gsems[0])
            cp0.wait()
            ocp0 = process(k0, 0)

            @pl.when(j < NPAIR - 1)
            def _():
                fetch(k0 + 2, 0)

            cp1.wait()
            ocp1 = process(k0 + 1, 1)
            for cp in ocp0 + ocp1:
                cp.wait()

    return body(xt, table, pos_table)


def kernel(x, emb_table, pos_table):
    xt = jnp.swapaxes(x, 0, 1).astype(jnp.int32)
    embt = jnp.pad(jnp.swapaxes(emb_table, 0, 1), ((0, 0), (0, VPAD - VOCAB)))
    in6 = embt.reshape(NTR, 8, NTILE, 128).transpose(0, 2, 1, 3)
    table = _sc_table_transpose(in6)
    out6 = _sc_embed(xt, table, pos_table)
    return out6.transpose(2, 4, 0, 1, 3).reshape(B, L, DIM)


# speedup vs baseline: 1.0414x; 1.0414x over previous
import functools

import jax
import jax.numpy as jnp
from jax import lax
from jax.experimental import pallas as pl
from jax.experimental.pallas import tpu as pltpu
from jax.experimental.pallas import tpu_sc as plsc

B = 4096
L = 200
DIM = 32
VOCAB = 1000000

NUM_CORES = 2
NUM_SUBCORES = 16
NW = NUM_CORES * NUM_SUBCORES

BCHUNK = 512
NCHUNK = B // BCHUNK
NBLK_TOTAL = L * NCHUNK
BLK_PER_W = NBLK_TOTAL // NW
NPAIR = BLK_PER_W // 2
NTR = DIM // 8
NTC = BCHUNK // 128
PITCH = DIM + 1

NTILE = 7816
VPAD = NTILE * 128
TBLK = 8
NTBLK = NTILE // TBLK
TP_PER_W = 31
OPITCH = DIM + 2


def _sc_table_transpose(in6):
    mesh = plsc.VectorSubcoreMesh(core_axis_name="c", subcore_axis_name="s")

    @functools.partial(
        pl.kernel,
        mesh=mesh,
        out_type=jax.ShapeDtypeStruct((VPAD, DIM), jnp.float32),
        compiler_params=pltpu.CompilerParams(
            use_tc_tiling_on_sc=False, needs_layout_passes=False),
        scratch_types=[
            pltpu.VMEM((2, NTR, TBLK, 8, 128), jnp.float32),
            pltpu.VMEM((TBLK * 128, OPITCH), jnp.float32),
            pltpu.SemaphoreType.DMA,
            pltpu.SemaphoreType.DMA,
        ],
    )
    def body(in_hbm, out_hbm, sbuf_v, obuf_v, sem0, sem1):
        wid = lax.axis_index("s") * NUM_CORES + lax.axis_index("c")
        sems = (sem0, sem1)

        def issue_in(bid, buf):
            @pl.when(bid < NTBLK)
            def _():
                for tr in range(NTR):
                    pltpu.async_copy(in_hbm.at[tr, pl.ds(bid * TBLK, TBLK)],
                                     sbuf_v.at[buf].at[tr], sems[buf])

        def wait_in(bid, buf):
            for tr in range(NTR):
                pltpu.make_async_copy(
                    in_hbm.at[tr, pl.ds(bid * TBLK, TBLK)],
                    sbuf_v.at[buf].at[tr], sems[buf]).wait()

        def do_block(bid, buf):
            wait_in(bid, buf)

            @plsc.parallel_loop(0, NTR * TBLK * 8, unroll=4)
            def _(it):
                tr = it // (TBLK * 8)
                tcr = (it // 8) % TBLK
                r = it % 8
                d = tr * 8 + r
                dsplat = jnp.zeros((16,), jnp.int32) + d
                for c8 in range(8):
                    c0 = c8 * 16
                    v = sbuf_v[buf, tr, tcr, r, pl.ds(c0, 16)]
                    tvec = tcr * 128 + c0 + lax.iota(jnp.int32, 16)
                    plsc.store_scatter(obuf_v, [tvec, dsplat], v)

            pltpu.sync_copy(
                obuf_v.at[pl.ds(0, TBLK * 128), pl.ds(0, DIM)],
                out_hbm.at[pl.ds(bid * TBLK * 128, TBLK * 128)])

        issue_in(wid, 0)

        for k in range(TP_PER_W):
            bid = wid + NW * k
            if k + 1 < TP_PER_W:
                issue_in(bid + NW, (k + 1) % 2)

            @pl.when(bid < NTBLK)
            def _():
                do_block(bid, k % 2)

    return body(in6)


def _sc_embed(xt, table, pos_table):
    mesh = plsc.VectorSubcoreMesh(core_axis_name="c", subcore_axis_name="s")

    @functools.partial(
        pl.kernel,
        mesh=mesh,
        out_type=jax.ShapeDtypeStruct((L, NTR, B // 128, 8, 128), jnp.float32),
        compiler_params=pltpu.CompilerParams(
            use_tc_tiling_on_sc=False, needs_layout_passes=False),
        scratch_types=[
            pltpu.VMEM((2, BCHUNK), jnp.int32),
            pltpu.VMEM((2, BCHUNK), jnp.float32),
            pltpu.VMEM((2, BCHUNK, DIM), jnp.float32),
            pltpu.VMEM((BCHUNK, PITCH), jnp.float32),
            pltpu.VMEM((2, NTR, NTC, 8, 128), jnp.float32),
            pltpu.VMEM((L, DIM), jnp.float32),
            pltpu.VMEM((DIM, 16), jnp.float32),
            pltpu.SemaphoreType.DMA,
            pltpu.SemaphoreType.DMA,
            pltpu.SemaphoreType.DMA,
            pltpu.SemaphoreType.DMA,
        ],
    )
    def body(x_hbm, emb_hbm, pos_hbm, out_hbm, tok_v, fm_v, rows_v,
             st_v, tbuf_v, pos_v, psplat_v, gsem0, gsem1, osem0, osem1):
        wid = lax.axis_index("s") * NUM_CORES + lax.axis_index("c")
        bid0 = wid * BLK_PER_W
        pltpu.sync_copy(pos_hbm.at[pl.ds(0, L)], pos_v)
        gsems = (gsem0, gsem1)
        osems = (osem0, osem1)

        def fetch(k, buf):
            bid = bid0 + k
            l = bid // NCHUNK
            b0 = (bid % NCHUNK) * BCHUNK
            pltpu.sync_copy(x_hbm.at[l, pl.ds(b0, BCHUNK)], tok_v.at[buf])

            @plsc.parallel_loop(0, BCHUNK, step=16, unroll=4)
            def _(i):
                iv = tok_v[buf, pl.ds(i, 16)]
                fm_v[buf, pl.ds(i, 16)] = jnp.where(
                    iv == jnp.int32(0), jnp.float32(0.0), jnp.float32(1.0))

            return pltpu.async_copy(
                emb_hbm.at[tok_v.at[buf]], rows_v.at[buf], gsems[buf])

        def process(k, buf):
            bid = bid0 + k
            l = bid // NCHUNK
            chunk = bid % NCHUNK

            lsplat = jnp.zeros((16,), jnp.int32) + l
            for d in range(DIM):
                psplat_v[d] = plsc.load_gather(
                    pos_v, [lsplat, jnp.full((16,), d, jnp.int32)])

            @plsc.parallel_loop(0, BCHUNK, step=4, unroll=4)
            def _(r):
                for u in range(4):
                    st_v[r + u, pl.ds(0, 16)] = rows_v[buf, r + u, pl.ds(0, 16)]
                    st_v[r + u, pl.ds(16, 16)] = (
                        rows_v[buf, r + u, pl.ds(16, 16)])

            @plsc.parallel_loop(0, BCHUNK // 16, unroll=2)
            def _(i16):
                i = i16 * 16
                rvec = i + lax.iota(jnp.int32, 16)
                fm = fm_v[buf, pl.ds(i, 16)]
                tc = i16 // 8
                c0 = (i16 % 8) * 16
                for d in range(DIM):
                    val = plsc.load_gather(
                        st_v, [rvec, jnp.full((16,), d, jnp.int32)])
                    tbuf_v[buf, d // 8, tc, d % 8, pl.ds(c0, 16)] = (
                        val + psplat_v[d] * fm)

            return [
                pltpu.async_copy(
                    tbuf_v.at[buf].at[tr],
                    out_hbm.at[l, tr, pl.ds(chunk * NTC, NTC)],
                    osems[buf])
                for tr in range(NTR)
            ]

        cp0 = fetch(0, 0)

        @pl.loop(0, NPAIR)
        def _(j):
            k0 = j * 2
            cp1 = fetch(k0 + 1, 1)
            cp0 = pltpu.make_async_copy(
                emb_hbm.at[tok_v.at[0]], rows_v.at[0], gsems[0])
            cp0.wait()
            ocp0 = process(k0, 0)

            @pl.when(j < NPAIR - 1)
            def _():
                fetch(k0 + 2, 0)

            cp1.wait()
            ocp1 = process(k0 + 1, 1)
            for cp in ocp0 + ocp1:
                cp.wait()

    return body(xt, table, pos_table)


def kernel(x, emb_table, pos_table):
    xt = jnp.swapaxes(x, 0, 1).astype(jnp.int32)
    embt = jnp.pad(jnp.swapaxes(emb_table, 0, 1), ((0, 0), (0, VPAD - VOCAB)))
    in6 = embt.reshape(NTR, 8, NTILE, 128).transpose(0, 2, 1, 3)
    table = _sc_table_transpose(in6)
    out6 = _sc_embed(xt, table, pos_table)
    return out6.transpose(2, 4, 0, 1, 3).reshape(B, L, DIM)
